# baseline (speedup 1.0000x reference)
import jax
import jax.numpy as jnp
from jax.experimental import pallas as pl
from jax.experimental.pallas import tpu as pltpu

VOCAB_SIZE = 25107
EMB_DIM = 100
CTX_LEN = 5
BATCH = 16
H1 = 300
H2 = 300

VOCAB_TILE = 2048
NUM_TILES = pl.cdiv(VOCAB_SIZE, VOCAB_TILE)


def _mlp_kernel(x_smem, emb_hbm, w1_ref, b1_ref, w2_ref, b2_ref, w3_ref,
                b3_ref, out_ref, ebuf, h2_ref, gsem):
    @pl.when(pl.program_id(0) == 0)
    def _():
        gathers = []
        for b in range(BATCH):
            for c in range(CTX_LEN):
                g = pltpu.make_async_copy(
                    emb_hbm.at[pl.ds(x_smem[b, c], 1), :],
                    ebuf.at[c, pl.ds(b, 1), :], gsem)
                g.start()
                gathers.append(g)
        for g in gathers:
            g.wait()

        h1 = b1_ref[...][None, :]
        for c in range(CTX_LEN):
            h1 = h1 + jnp.dot(ebuf[c], w1_ref[c],
                              preferred_element_type=jnp.float32)
        h1 = jnp.maximum(h1, 0.0)
        h2_ref[...] = jnp.maximum(
            jnp.dot(h1, w2_ref[...],
                    preferred_element_type=jnp.float32)
            + b2_ref[...][None, :], 0.0)

    out_ref[...] = jnp.dot(h2_ref[...], w3_ref[...],
                           preferred_element_type=jnp.float32) \
        + b3_ref[...][None, :]


def kernel(x, emb, W1, b1, W2, b2, W3, b3):
    return pl.pallas_call(
        _mlp_kernel,
        grid=(NUM_TILES,),
        in_specs=[
            pl.BlockSpec(memory_space=pltpu.SMEM),
            pl.BlockSpec(memory_space=pl.ANY),
            pl.BlockSpec((CTX_LEN, EMB_DIM, H1), lambda i: (0, 0, 0)),
            pl.BlockSpec((H1,), lambda i: (0,)),
            pl.BlockSpec((H1, H2), lambda i: (0, 0)),
            pl.BlockSpec((H2,), lambda i: (0,)),
            pl.BlockSpec((H2, VOCAB_TILE), lambda i: (0, i)),
            pl.BlockSpec((VOCAB_TILE,), lambda i: (i,)),
        ],
        out_specs=pl.BlockSpec((BATCH, VOCAB_TILE), lambda i: (0, i)),
        out_shape=jax.ShapeDtypeStruct((BATCH, VOCAB_SIZE), jnp.float32),
        scratch_shapes=[
            pltpu.VMEM((CTX_LEN, BATCH, EMB_DIM), jnp.float32),
            pltpu.VMEM((BATCH, H2), jnp.float32),
            pltpu.SemaphoreType.DMA,
        ],
    )(x, emb, W1.reshape(CTX_LEN, EMB_DIM, H1), b1, W2, b2, W3, b3)


# speedup vs baseline: 1.0833x; 1.0833x over previous
import jax
import jax.numpy as jnp
from jax.experimental import pallas as pl
from jax.experimental.pallas import tpu as pltpu

VOCAB_SIZE = 25107
EMB_DIM = 100
CTX_LEN = 5
BATCH = 16
H1 = 300
H2 = 300

VOCAB_TILE = 2048
NUM_FULL_TILES = VOCAB_SIZE // VOCAB_TILE
TAIL = VOCAB_SIZE - NUM_FULL_TILES * VOCAB_TILE
NBUF = 4


def _mlp_kernel(x_smem, emb_hbm, w1_ref, b1_ref, w2_ref, b2_ref, w3_hbm,
                b3_ref, out_ref, gbuf, bufs, tail_buf, gsem, sems, tail_sem):
    def start_fetch(i):
        pltpu.make_async_copy(
            w3_hbm.at[:, pl.ds(i * VOCAB_TILE, VOCAB_TILE)],
            bufs.at[i % NBUF],
            sems.at[i % NBUF],
        ).start()

    tail_copy = pltpu.make_async_copy(
        w3_hbm.at[:, pl.ds(NUM_FULL_TILES * VOCAB_TILE, TAIL)],
        tail_buf,
        tail_sem,
    )
    tail_copy.start()
    for i in range(NBUF):
        start_fetch(i)

    gathers = []
    for b in range(BATCH):
        for c in range(CTX_LEN):
            group = (x_smem[b, c] // 8) * 8
            g = pltpu.make_async_copy(
                emb_hbm.at[pl.ds(group, 8), :],
                gbuf.at[c, b], gsem)
            g.start()
            gathers.append(g)
    for g in gathers:
        g.wait()

    h1 = b1_ref[...][None, :]
    for c in range(CTX_LEN):
        rows = [gbuf[c, b, pl.ds(x_smem[b, c] % 8, 1), :]
                for b in range(BATCH)]
        e_c = jnp.concatenate(rows, axis=0)
        h1 = h1 + jnp.dot(e_c, w1_ref[c],
                          preferred_element_type=jnp.float32)
    h1 = jnp.maximum(h1, 0.0)
    h2 = jnp.maximum(
        jnp.dot(h1, w2_ref[...],
                preferred_element_type=jnp.float32) + b2_ref[...][None, :],
        0.0)

    for i in range(NUM_FULL_TILES):
        pltpu.make_async_copy(
            w3_hbm.at[:, pl.ds(i * VOCAB_TILE, VOCAB_TILE)],
            bufs.at[i % NBUF],
            sems.at[i % NBUF],
        ).wait()
        tile = jnp.dot(h2, bufs[i % NBUF],
                       preferred_element_type=jnp.float32)
        if i + NBUF < NUM_FULL_TILES:
            start_fetch(i + NBUF)
        out_ref[:, pl.ds(i * VOCAB_TILE, VOCAB_TILE)] = (
            tile + b3_ref[pl.ds(i * VOCAB_TILE, VOCAB_TILE)][None, :])

    tail_copy.wait()
    base = NUM_FULL_TILES * VOCAB_TILE
    tail = jnp.dot(h2, tail_buf[...], preferred_element_type=jnp.float32)
    out_ref[:, pl.ds(base, TAIL)] = tail + b3_ref[pl.ds(base, TAIL)][None, :]


def kernel(x, emb, W1, b1, W2, b2, W3, b3):
    return pl.pallas_call(
        _mlp_kernel,
        in_specs=[
            pl.BlockSpec(memory_space=pltpu.SMEM),
            pl.BlockSpec(memory_space=pl.ANY),
            pl.BlockSpec(memory_space=pltpu.VMEM),
            pl.BlockSpec(memory_space=pltpu.VMEM),
            pl.BlockSpec(memory_space=pltpu.VMEM),
            pl.BlockSpec(memory_space=pltpu.VMEM),
            pl.BlockSpec(memory_space=pl.ANY),
            pl.BlockSpec(memory_space=pltpu.VMEM),
        ],
        out_specs=pl.BlockSpec(memory_space=pltpu.VMEM),
        out_shape=jax.ShapeDtypeStruct((BATCH, VOCAB_SIZE), jnp.float32),
        scratch_shapes=[
            pltpu.VMEM((CTX_LEN, BATCH, 8, EMB_DIM), jnp.float32),
            pltpu.VMEM((NBUF, H2, VOCAB_TILE), jnp.float32),
            pltpu.VMEM((H2, TAIL), jnp.float32),
            pltpu.SemaphoreType.DMA,
            pltpu.SemaphoreType.DMA((NBUF,)),
            pltpu.SemaphoreType.DMA,
        ],
    )(x, emb, W1.reshape(CTX_LEN, EMB_DIM, H1), b1, W2, b2, W3, b3)
